# baseline (speedup 1.0000x reference)
import functools

import jax
import jax.numpy as jnp
from jax import lax
from jax.experimental import pallas as pl
from jax.experimental.pallas import tpu as pltpu
from jax.experimental.pallas import tpu_sc as plsc

N = 10000
E = 320000
D = 128
H = 128
MH = 128

NC = 2
NS = 16
NW = NC * NS
EPW = E // NW
CH = 80
NCHUNK = EPW // CH
SCH = 5
NSUP = NCHUNK // SCH
NP = 10240
RPT = NP // NS
ZR = 32

BN = 1000
GRID = N // BN


def _premask_body(row_h, col_h, vals_h, t_h, out,
                  rowb, colb, valb, vsb, vdb, t_v, sem):
    cid = lax.axis_index("c")
    sid = lax.axis_index("s")
    wid = cid * NS + sid
    pltpu.sync_copy(t_h, t_v)

    def chunk(i, carry):
        base = wid * EPW + i * CH
        c1 = pltpu.async_copy(row_h.at[pl.ds(base, CH)], rowb, sem)
        c2 = pltpu.async_copy(col_h.at[pl.ds(base, CH)], colb, sem)
        c3 = pltpu.async_copy(vals_h.at[pl.ds(base, CH)], valb, sem)
        c1.wait()
        c2.wait()
        c3.wait()
        for s in range(CH // 16):
            sl = pl.ds(s * 16, 16)
            v16 = valb[sl]
            tr = plsc.load_gather(t_v, [rowb[sl]])
            tc = plsc.load_gather(t_v, [colb[sl]])
            vs = jnp.where(tr == tc, v16, 0.0)
            vsb[sl] = vs
            vdb[sl] = v16 - vs
        pltpu.sync_copy(valb, out.at[pl.ds(base, CH)])
        pltpu.sync_copy(vsb, out.at[pl.ds(E + base, CH)])
        pltpu.sync_copy(vdb, out.at[pl.ds(2 * E + base, CH)])
        return carry
    lax.fori_loop(0, NCHUNK, chunk, 0)


def _premask(row2d, col2d, vals2d, t):
    return pl.kernel(
        _premask_body,
        out_type=jax.ShapeDtypeStruct((3 * E,), jnp.float32),
        mesh=plsc.VectorSubcoreMesh(core_axis_name="c", subcore_axis_name="s",
                                    num_cores=NC, num_subcores=NS),
        compiler_params=pltpu.CompilerParams(needs_layout_passes=False),
        scratch_types=[
            pltpu.VMEM((CH,), jnp.int32),
            pltpu.VMEM((CH,), jnp.int32),
            pltpu.VMEM((CH,), jnp.float32),
            pltpu.VMEM((CH,), jnp.float32),
            pltpu.VMEM((CH,), jnp.float32),
            pltpu.VMEM((N,), jnp.float32),
            pltpu.SemaphoreType.DMA,
        ],
    )(row2d, col2d, vals2d, t)


def _spmm_body(ha, hc, hf, row_h, col_h, vals3_h, out,
               rowb, colb, valb, rows0, rows1, zbuf, acc,
               sem_s0, sem_s1, sem_s2, sem_s3, sem_g0, sem_g1, sem_c0, sem_c1):
    cid = lax.axis_index("c")
    sid = lax.axis_index("s")
    wid = cid * NS + sid
    rows = (rows0, rows1)
    sem_s = (sem_s0, sem_s1, sem_s2, sem_s3)
    sem_g = (sem_g0, sem_g1)
    sem_c = (sem_c0, sem_c1)

    def zrow(r, carry):
        for j in range(8):
            zbuf[r, pl.ds(j * 16, 16)] = jnp.zeros((16,), jnp.float32)
        return carry
    lax.fori_loop(0, ZR, zrow, 0)

    def stage(b, c, s):
        base = wid * EPW + c * CH
        pltpu.async_copy(row_h.at[pl.ds(base, CH)], rowb.at[s], sem_s[s])
        pltpu.async_copy(col_h.at[pl.ds(base, CH)], colb.at[s], sem_s[s])
        pltpu.async_copy(vals3_h.at[pl.ds(b * E + base, CH)], valb.at[s],
                         sem_s[s])

    def wait_stage(c, s):
        base = wid * EPW + c * CH
        for _ in range(3):
            pltpu.make_async_copy(row_h.at[pl.ds(base, CH)], rowb.at[s],
                                  sem_s[s]).wait()

    def drain_scatter(r, s):
        pltpu.make_async_copy(rows[r], acc.at[rowb.at[s]], sem_c[r]).wait()

    def do_chunk(h_h, b, c, s, r, drain, prefetch):
        wait_stage(c, s)
        if drain:
            drain_scatter(r, (s + 2) % 4)
        pltpu.async_copy(h_h.at[colb.at[s]], rows[r], sem_g[r])
        if prefetch:
            stage(b, c + 2, (s + 2) % 4)
        pltpu.make_async_copy(h_h.at[colb.at[s]], rows[r], sem_g[r]).wait()
        rv = rows[r]

        def edge8(i8, c2_):
            eb = i8 * 8
            sc = jnp.zeros((16,), jnp.int32) + s
            bvs = [plsc.load_gather(valb, [sc, jnp.zeros((16,), jnp.int32)
                                           + (eb + ee)])
                   for ee in range(8)]
            for ee in range(8):
                for jj in range(8):
                    sl2 = pl.ds(jj * 16, 16)
                    rv[eb + ee, sl2] = rv[eb + ee, sl2] * bvs[ee]
            return c2_
        lax.fori_loop(0, CH // 8, edge8, 0)
        pltpu.async_copy(rv, acc.at[rowb.at[s]], sem_c[r], add=True)

    for b, h_h in enumerate((ha, hc, hf)):
        plsc.subcore_barrier()
        for k in range(RPT // ZR):
            pltpu.sync_copy(zbuf, acc.at[pl.ds(sid * RPT + k * ZR, ZR)])
        plsc.subcore_barrier()

        stage(b, 0, 0)
        stage(b, 1, 1)
        do_chunk(h_h, b, 0, 0, 0, drain=False, prefetch=True)
        do_chunk(h_h, b, 1, 1, 1, drain=False, prefetch=True)

        def quad(q, carry):
            c0 = 2 + 4 * q
            for u in range(4):
                do_chunk(h_h, b, c0 + u, (2 + u) % 4, u % 2,
                         drain=True, prefetch=True)
            return carry
        lax.fori_loop(0, (NCHUNK - 5) // 4, quad, 0)

        do_chunk(h_h, b, NCHUNK - 3, 2, 0, drain=True, prefetch=True)
        do_chunk(h_h, b, NCHUNK - 2, 3, 1, drain=True, prefetch=False)
        do_chunk(h_h, b, NCHUNK - 1, 0, 0, drain=True, prefetch=False)
        drain_scatter(1, 3)
        drain_scatter(0, 0)

        plsc.subcore_barrier()
        pltpu.sync_copy(acc.at[pl.ds(sid * RPT, RPT)],
                        out.at[b, cid, pl.ds(sid * RPT, RPT)])


def _spmm_layer(ha, hc, hf, row2d, col2d, vals3):
    return pl.kernel(
        _spmm_body,
        out_type=jax.ShapeDtypeStruct((3, NC, NP, H), jnp.float32),
        mesh=plsc.VectorSubcoreMesh(core_axis_name="c", subcore_axis_name="s",
                                    num_cores=NC, num_subcores=NS),
        compiler_params=pltpu.CompilerParams(needs_layout_passes=False),
        scratch_types=[
            pltpu.VMEM((4, CH), jnp.int32),
            pltpu.VMEM((4, CH), jnp.int32),
            pltpu.VMEM((4, CH), jnp.float32),
            pltpu.VMEM((CH, H), jnp.float32),
            pltpu.VMEM((CH, H), jnp.float32),
            pltpu.VMEM((ZR, H), jnp.float32),
            pltpu.VMEM_SHARED((NP, H), jnp.float32),
            pltpu.SemaphoreType.DMA,
            pltpu.SemaphoreType.DMA,
            pltpu.SemaphoreType.DMA,
            pltpu.SemaphoreType.DMA,
            pltpu.SemaphoreType.DMA,
            pltpu.SemaphoreType.DMA,
            pltpu.SemaphoreType.DMA,
            pltpu.SemaphoreType.DMA,
        ],
    )(ha, hc, hf, row2d, col2d, vals3)


def _k1_body(x_ref, dw1, db1, dw2, db2, aw1, cw1, fw1, ha_ref, hc_ref, hf_ref):
    xb = x_ref[...]
    h = jnp.maximum(jnp.dot(xb, dw1[...], preferred_element_type=jnp.float32)
                    + db1[...], 0.0)
    m = jax.nn.sigmoid(jnp.dot(h, dw2[...], preferred_element_type=jnp.float32)
                       + db2[...])
    xa = xb * m
    xc = xb - xa
    ha_ref[...] = jnp.dot(xa, aw1[...], preferred_element_type=jnp.float32)
    hc_ref[...] = jnp.dot(xc, cw1[...], preferred_element_type=jnp.float32)
    hf_ref[...] = jnp.dot(xc, fw1[...], preferred_element_type=jnp.float32)


def _k1(x, p):
    full = lambda s: pl.BlockSpec(s, lambda i: (0, 0))
    row = pl.BlockSpec((BN, D), lambda i: (i, 0))
    return pl.pallas_call(
        _k1_body,
        grid=(GRID,),
        in_specs=[row, full((D, 256)), full((1, 256)), full((256, D)),
                  full((1, D)), full((D, H)), full((D, H)), full((D, H))],
        out_specs=[pl.BlockSpec((BN, H), lambda i: (i, 0))] * 3,
        out_shape=[jax.ShapeDtypeStruct((N, H), jnp.float32)] * 3,
    )(x, p['d_w1'], p['d_b1'].reshape(1, -1), p['d_w2'],
      p['d_b2'].reshape(1, -1), p['adj_w1'], p['conf_w1'], p['cf_w1'])


def _k2_body(p_ref, ba, bc, bf, wa, wc, wf, oa, oc, of):
    for b, (bb, ww, oo) in enumerate(((ba, wa, oa), (bc, wc, oc), (bf, wf, of))):
        s = p_ref[b, 0] + p_ref[b, 1]
        h1 = jnp.maximum(s + bb[...], 0.0)
        oo[...] = jnp.dot(h1, ww[...], preferred_element_type=jnp.float32)


def _k2(psum, p):
    full = lambda s: pl.BlockSpec(s, lambda i: (0, 0))
    return pl.pallas_call(
        _k2_body,
        grid=(GRID,),
        in_specs=[pl.BlockSpec((3, NC, BN, H), lambda i: (0, 0, i, 0)),
                  full((1, H)), full((1, H)), full((1, H)),
                  full((H, H)), full((H, H)), full((H, H))],
        out_specs=[pl.BlockSpec((BN, H), lambda i: (i, 0))] * 3,
        out_shape=[jax.ShapeDtypeStruct((N, H), jnp.float32)] * 3,
    )(psum, p['adj_b1'].reshape(1, -1), p['conf_b1'].reshape(1, -1),
      p['cf_b1'].reshape(1, -1), p['adj_w2'], p['conf_w2'], p['cf_w2'])


def _k3_body(p_ref, ba, bc, bf, m0w1, m0b1, m0w2, m0b2, m1w1, m1b1, m1w2,
             m1b2, pw, pb, tpw, tpb, mw, mb, t_ref,
             e_ref, mu0_ref, mu1_ref, tau_ref, adj_ref, tl_ref, ml_ref):
    i = pl.program_id(0)
    adj = jnp.maximum(p_ref[0, 0] + p_ref[0, 1] + ba[...], 0.0)
    conf = jnp.maximum(p_ref[1, 0] + p_ref[1, 1] + bc[...], 0.0)
    cf = jnp.maximum(p_ref[2, 0] + p_ref[2, 1] + bf[...], 0.0)
    adj_ref[...] = adj

    def cat_mm(w_ref):
        return (jnp.dot(adj, w_ref[0:H, :], preferred_element_type=jnp.float32)
                + jnp.dot(conf, w_ref[H:2 * H, :],
                          preferred_element_type=jnp.float32))

    h0 = jnp.maximum(cat_mm(m0w1) + m0b1[...], 0.0)
    mu0 = jnp.dot(h0, m0w2[...], preferred_element_type=jnp.float32) + m0b2[...]
    h1 = jnp.maximum(cat_mm(m1w1) + m1b1[...], 0.0)
    mu1 = jnp.dot(h1, m1w2[...], preferred_element_type=jnp.float32) + m1b2[...]
    ev = jax.nn.sigmoid(cat_mm(pw) + pb[...])
    e_ref[...] = ev
    mu0_ref[...] = mu0
    mu1_ref[...] = mu1
    tau_ref[...] = mu1 - mu0

    tp = jax.nn.sigmoid(jnp.dot(conf, tpw[...],
                                preferred_element_type=jnp.float32) + tpb[...])
    eps = 1e-7
    tpc = jnp.clip(tp, eps, 1.0 - eps)
    tb = t_ref[...]
    bce = -(tb * jnp.log(tpc) + (1.0 - tb) * jnp.log(1.0 - tpc))
    mapped = (jnp.dot(conf, mw[...], preferred_element_type=jnp.float32)
              + mb[...])
    se = (mapped - cf) ** 2

    @pl.when(i == 0)
    def _():
        tl_ref[...] = jnp.zeros((1, 1), jnp.float32)
        ml_ref[...] = jnp.zeros((1, 1), jnp.float32)
    tl_ref[...] += jnp.sum(bce).reshape(1, 1)
    ml_ref[...] += jnp.sum(se).reshape(1, 1)


def _k3(psum, t2, p):
    full = lambda s: pl.BlockSpec(s, lambda i: (0, 0))
    col = lambda: pl.BlockSpec((BN, 1), lambda i: (i, 0))
    return pl.pallas_call(
        _k3_body,
        grid=(GRID,),
        in_specs=[pl.BlockSpec((3, NC, BN, H), lambda i: (0, 0, i, 0)),
                  full((1, H)), full((1, H)), full((1, H)),
                  full((2 * H, MH)), full((1, MH)), full((MH, 1)), full((1, 1)),
                  full((2 * H, MH)), full((1, MH)), full((MH, 1)), full((1, 1)),
                  full((2 * H, 1)), full((1, 1)),
                  full((H, 1)), full((1, 1)),
                  full((H, H)), full((1, H)),
                  col()],
        out_specs=[col(), col(), col(), col(),
                   pl.BlockSpec((BN, H), lambda i: (i, 0)),
                   full((1, 1)), full((1, 1))],
        out_shape=[jax.ShapeDtypeStruct((N, 1), jnp.float32)] * 4
        + [jax.ShapeDtypeStruct((N, H), jnp.float32),
           jax.ShapeDtypeStruct((1, 1), jnp.float32),
           jax.ShapeDtypeStruct((1, 1), jnp.float32)],
    )(psum, p['adj_b2'].reshape(1, -1), p['conf_b2'].reshape(1, -1),
      p['cf_b2'].reshape(1, -1),
      p['mu0_w1'], p['mu0_b1'].reshape(1, -1), p['mu0_w2'],
      p['mu0_b2'].reshape(1, -1),
      p['mu1_w1'], p['mu1_b1'].reshape(1, -1), p['mu1_w2'],
      p['mu1_b2'].reshape(1, -1),
      p['prop_w'], p['prop_b'].reshape(1, -1),
      p['tp_w'], p['tp_b'].reshape(1, -1),
      p['map_w'], p['map_b'].reshape(1, -1),
      t2)


def kernel(x, edge_index, adj_vals, t, params):
    p = params
    row1 = edge_index[0]
    col1 = edge_index[1]

    vals3 = _premask(row1, col1, adj_vals, t)
    ha1, hc1, hf1 = _k1(x, p)
    p1 = _spmm_layer(ha1, hc1, hf1, row1, col1, vals3)
    ha2, hc2, hf2 = _k2(p1, p)
    p2 = _spmm_layer(ha2, hc2, hf2, row1, col1, vals3)
    e2, mu02, mu12, tau2, adj_rep, tl, ml = _k3(p2, t.reshape(N, 1), p)

    e = e2[:, 0]
    mu0 = mu02[:, 0]
    mu1 = mu12[:, 0]
    tau = tau2[:, 0]
    treat_loss = tl[0, 0] / N
    map_loss = ml[0, 0] / (N * H)
    return (e, mu0, mu1, tau, adj_rep, treat_loss, map_loss)


# speedup vs baseline: 1.5373x; 1.5373x over previous
import functools

import jax
import jax.numpy as jnp
from jax import lax
from jax.experimental import pallas as pl
from jax.experimental.pallas import tpu as pltpu
from jax.experimental.pallas import tpu_sc as plsc

N = 10000
E = 320000
D = 128
H = 128
MH = 128

NC = 2
NS = 16
NW = NC * NS
EPW = E // NW
CH = 80
NCHUNK = EPW // CH
SCH = 5
NSUP = NCHUNK // SCH
NP = 10240
RPT = NP // NS
ZR = 32

BN = 1000
GRID = N // BN


def _premask_body(row_h, col_h, vals_h, t_h, out,
                  rowb, colb, valb, vsb, vdb, t_v, sem):
    cid = lax.axis_index("c")
    sid = lax.axis_index("s")
    wid = cid * NS + sid
    pltpu.sync_copy(t_h, t_v)

    def chunk(i, carry):
        base = wid * EPW + i * CH
        c1 = pltpu.async_copy(row_h.at[pl.ds(base, CH)], rowb, sem)
        c2 = pltpu.async_copy(col_h.at[pl.ds(base, CH)], colb, sem)
        c3 = pltpu.async_copy(vals_h.at[pl.ds(base, CH)], valb, sem)
        c1.wait()
        c2.wait()
        c3.wait()
        for s in range(CH // 16):
            sl = pl.ds(s * 16, 16)
            v16 = valb[sl]
            tr = plsc.load_gather(t_v, [rowb[sl]])
            tc = plsc.load_gather(t_v, [colb[sl]])
            vs = jnp.where(tr == tc, v16, 0.0)
            vsb[sl] = vs
            vdb[sl] = v16 - vs
        pltpu.sync_copy(valb, out.at[pl.ds(base, CH)])
        pltpu.sync_copy(vsb, out.at[pl.ds(E + base, CH)])
        pltpu.sync_copy(vdb, out.at[pl.ds(2 * E + base, CH)])
        return carry
    lax.fori_loop(0, NCHUNK, chunk, 0)


def _premask(row2d, col2d, vals2d, t):
    return pl.kernel(
        _premask_body,
        out_type=jax.ShapeDtypeStruct((3 * E,), jnp.float32),
        mesh=plsc.VectorSubcoreMesh(core_axis_name="c", subcore_axis_name="s",
                                    num_cores=NC, num_subcores=NS),
        compiler_params=pltpu.CompilerParams(needs_layout_passes=False),
        scratch_types=[
            pltpu.VMEM((CH,), jnp.int32),
            pltpu.VMEM((CH,), jnp.int32),
            pltpu.VMEM((CH,), jnp.float32),
            pltpu.VMEM((CH,), jnp.float32),
            pltpu.VMEM((CH,), jnp.float32),
            pltpu.VMEM((N,), jnp.float32),
            pltpu.SemaphoreType.DMA,
        ],
    )(row2d, col2d, vals2d, t)


def _spmm_body(ha, hc, hf, row_h, col_h, vals3_h, out,
               rowb, colb, valb, rows0, rows1, zbuf, acc,
               sem_s0, sem_s1, sem_s2, sem_s3, sem_g0, sem_g1, sem_c0, sem_c1):
    cid = lax.axis_index("c")
    sid = lax.axis_index("s")
    wid = cid * NS + sid
    rows = (rows0, rows1)
    sem_s = (sem_s0, sem_s1, sem_s2, sem_s3)
    sem_g = (sem_g0, sem_g1)
    sem_c = (sem_c0, sem_c1)

    def zrow(r, carry):
        for j in range(8):
            zbuf[r, pl.ds(j * 16, 16)] = jnp.zeros((16,), jnp.float32)
        return carry
    lax.fori_loop(0, ZR, zrow, 0)

    def stage(b, c, s):
        base = wid * EPW + c * CH
        pltpu.async_copy(row_h.at[pl.ds(base, CH)], rowb.at[s], sem_s[s])
        pltpu.async_copy(col_h.at[pl.ds(base, CH)], colb.at[s], sem_s[s])
        pltpu.async_copy(vals3_h.at[pl.ds(b * E + base, CH)], valb.at[s],
                         sem_s[s])

    def wait_stage(c, s):
        base = wid * EPW + c * CH
        for _ in range(3):
            pltpu.make_async_copy(row_h.at[pl.ds(base, CH)], rowb.at[s],
                                  sem_s[s]).wait()

    def drain_scatter(r, s):
        pltpu.make_async_copy(rows[r], acc.at[rowb.at[s]], sem_c[r]).wait()

    def do_chunk(h_h, b, c, s, r, drain, more, pre2=True):
        pltpu.make_async_copy(h_h.at[colb.at[s]], rows[r], sem_g[r]).wait()
        if drain:
            drain_scatter(1 - r, (s + 3) % 4)
        if more:
            wait_stage(c + 1, (s + 1) % 4)
            pltpu.async_copy(h_h.at[colb.at[(s + 1) % 4]], rows[1 - r],
                             sem_g[1 - r])
            if pre2:
                stage(b, c + 2, (s + 2) % 4)
        rv = rows[r]

        def edge(e, c2_):
            ei = jnp.zeros((16,), jnp.int32) + e
            bv = plsc.load_gather(valb,
                                  [jnp.zeros((16,), jnp.int32) + s, ei])
            for jj in range(8):
                sl2 = pl.ds(jj * 16, 16)
                rv[e, sl2] = rv[e, sl2] * bv
            return c2_
        lax.fori_loop(0, CH, edge, 0)
        pltpu.async_copy(rv, acc.at[rowb.at[s]], sem_c[r], add=True)

    for b, h_h in enumerate((ha, hc, hf)):
        plsc.subcore_barrier()
        for k in range(RPT // ZR):
            pltpu.sync_copy(zbuf, acc.at[pl.ds(sid * RPT + k * ZR, ZR)])
        plsc.subcore_barrier()

        stage(b, 0, 0)
        stage(b, 1, 1)
        wait_stage(0, 0)
        pltpu.async_copy(h_h.at[colb.at[0]], rows0, sem_g0)
        do_chunk(h_h, b, 0, 0, 0, drain=False, more=True)
        do_chunk(h_h, b, 1, 1, 1, drain=True, more=True)

        def quad(q, carry):
            c0 = 2 + 4 * q
            for u in range(4):
                do_chunk(h_h, b, c0 + u, (2 + u) % 4, u % 2,
                         drain=True, more=True)
            return carry
        lax.fori_loop(0, (NCHUNK - 5) // 4, quad, 0)

        do_chunk(h_h, b, NCHUNK - 3, 2, 0, drain=True, more=True, pre2=True)
        do_chunk(h_h, b, NCHUNK - 2, 3, 1, drain=True, more=True, pre2=False)
        do_chunk(h_h, b, NCHUNK - 1, 0, 0, drain=True, more=False)
        drain_scatter(0, 0)

        plsc.subcore_barrier()
        pltpu.sync_copy(acc.at[pl.ds(sid * RPT, RPT)],
                        out.at[b, cid, pl.ds(sid * RPT, RPT)])


def _spmm_layer(ha, hc, hf, row2d, col2d, vals3):
    return pl.kernel(
        _spmm_body,
        out_type=jax.ShapeDtypeStruct((3, NC, NP, H), jnp.float32),
        mesh=plsc.VectorSubcoreMesh(core_axis_name="c", subcore_axis_name="s",
                                    num_cores=NC, num_subcores=NS),
        compiler_params=pltpu.CompilerParams(needs_layout_passes=False),
        scratch_types=[
            pltpu.VMEM((4, CH), jnp.int32),
            pltpu.VMEM((4, CH), jnp.int32),
            pltpu.VMEM((4, CH), jnp.float32),
            pltpu.VMEM((CH, H), jnp.float32),
            pltpu.VMEM((CH, H), jnp.float32),
            pltpu.VMEM((ZR, H), jnp.float32),
            pltpu.VMEM_SHARED((NP, H), jnp.float32),
            pltpu.SemaphoreType.DMA,
            pltpu.SemaphoreType.DMA,
            pltpu.SemaphoreType.DMA,
            pltpu.SemaphoreType.DMA,
            pltpu.SemaphoreType.DMA,
            pltpu.SemaphoreType.DMA,
            pltpu.SemaphoreType.DMA,
            pltpu.SemaphoreType.DMA,
        ],
    )(ha, hc, hf, row2d, col2d, vals3)


def _k1_body(x_ref, dw1, db1, dw2, db2, aw1, cw1, fw1, ha_ref, hc_ref, hf_ref):
    xb = x_ref[...]
    h = jnp.maximum(jnp.dot(xb, dw1[...], preferred_element_type=jnp.float32)
                    + db1[...], 0.0)
    m = jax.nn.sigmoid(jnp.dot(h, dw2[...], preferred_element_type=jnp.float32)
                       + db2[...])
    xa = xb * m
    xc = xb - xa
    ha_ref[...] = jnp.dot(xa, aw1[...], preferred_element_type=jnp.float32)
    hc_ref[...] = jnp.dot(xc, cw1[...], preferred_element_type=jnp.float32)
    hf_ref[...] = jnp.dot(xc, fw1[...], preferred_element_type=jnp.float32)


def _k1(x, p):
    full = lambda s: pl.BlockSpec(s, lambda i: (0, 0))
    row = pl.BlockSpec((BN, D), lambda i: (i, 0))
    return pl.pallas_call(
        _k1_body,
        grid=(GRID,),
        in_specs=[row, full((D, 256)), full((1, 256)), full((256, D)),
                  full((1, D)), full((D, H)), full((D, H)), full((D, H))],
        out_specs=[pl.BlockSpec((BN, H), lambda i: (i, 0))] * 3,
        out_shape=[jax.ShapeDtypeStruct((N, H), jnp.float32)] * 3,
    )(x, p['d_w1'], p['d_b1'].reshape(1, -1), p['d_w2'],
      p['d_b2'].reshape(1, -1), p['adj_w1'], p['conf_w1'], p['cf_w1'])


def _k2_body(p_ref, ba, bc, bf, wa, wc, wf, oa, oc, of):
    for b, (bb, ww, oo) in enumerate(((ba, wa, oa), (bc, wc, oc), (bf, wf, of))):
        s = p_ref[b, 0] + p_ref[b, 1]
        h1 = jnp.maximum(s + bb[...], 0.0)
        oo[...] = jnp.dot(h1, ww[...], preferred_element_type=jnp.float32)


def _k2(psum, p):
    full = lambda s: pl.BlockSpec(s, lambda i: (0, 0))
    return pl.pallas_call(
        _k2_body,
        grid=(GRID,),
        in_specs=[pl.BlockSpec((3, NC, BN, H), lambda i: (0, 0, i, 0)),
                  full((1, H)), full((1, H)), full((1, H)),
                  full((H, H)), full((H, H)), full((H, H))],
        out_specs=[pl.BlockSpec((BN, H), lambda i: (i, 0))] * 3,
        out_shape=[jax.ShapeDtypeStruct((N, H), jnp.float32)] * 3,
    )(psum, p['adj_b1'].reshape(1, -1), p['conf_b1'].reshape(1, -1),
      p['cf_b1'].reshape(1, -1), p['adj_w2'], p['conf_w2'], p['cf_w2'])


def _k3_body(p_ref, ba, bc, bf, m0w1, m0b1, m0w2, m0b2, m1w1, m1b1, m1w2,
             m1b2, pw, pb, tpw, tpb, mw, mb, t_ref,
             e_ref, mu0_ref, mu1_ref, tau_ref, adj_ref, tl_ref, ml_ref):
    i = pl.program_id(0)
    adj = jnp.maximum(p_ref[0, 0] + p_ref[0, 1] + ba[...], 0.0)
    conf = jnp.maximum(p_ref[1, 0] + p_ref[1, 1] + bc[...], 0.0)
    cf = jnp.maximum(p_ref[2, 0] + p_ref[2, 1] + bf[...], 0.0)
    adj_ref[...] = adj

    def cat_mm(w_ref):
        return (jnp.dot(adj, w_ref[0:H, :], preferred_element_type=jnp.float32)
                + jnp.dot(conf, w_ref[H:2 * H, :],
                          preferred_element_type=jnp.float32))

    h0 = jnp.maximum(cat_mm(m0w1) + m0b1[...], 0.0)
    mu0 = jnp.dot(h0, m0w2[...], preferred_element_type=jnp.float32) + m0b2[...]
    h1 = jnp.maximum(cat_mm(m1w1) + m1b1[...], 0.0)
    mu1 = jnp.dot(h1, m1w2[...], preferred_element_type=jnp.float32) + m1b2[...]
    ev = jax.nn.sigmoid(cat_mm(pw) + pb[...])
    e_ref[...] = ev
    mu0_ref[...] = mu0
    mu1_ref[...] = mu1
    tau_ref[...] = mu1 - mu0

    tp = jax.nn.sigmoid(jnp.dot(conf, tpw[...],
                                preferred_element_type=jnp.float32) + tpb[...])
    eps = 1e-7
    tpc = jnp.clip(tp, eps, 1.0 - eps)
    tb = t_ref[...]
    bce = -(tb * jnp.log(tpc) + (1.0 - tb) * jnp.log(1.0 - tpc))
    mapped = (jnp.dot(conf, mw[...], preferred_element_type=jnp.float32)
              + mb[...])
    se = (mapped - cf) ** 2

    @pl.when(i == 0)
    def _():
        tl_ref[...] = jnp.zeros((1, 1), jnp.float32)
        ml_ref[...] = jnp.zeros((1, 1), jnp.float32)
    tl_ref[...] += jnp.sum(bce).reshape(1, 1)
    ml_ref[...] += jnp.sum(se).reshape(1, 1)


def _k3(psum, t2, p):
    full = lambda s: pl.BlockSpec(s, lambda i: (0, 0))
    col = lambda: pl.BlockSpec((BN, 1), lambda i: (i, 0))
    return pl.pallas_call(
        _k3_body,
        grid=(GRID,),
        in_specs=[pl.BlockSpec((3, NC, BN, H), lambda i: (0, 0, i, 0)),
                  full((1, H)), full((1, H)), full((1, H)),
                  full((2 * H, MH)), full((1, MH)), full((MH, 1)), full((1, 1)),
                  full((2 * H, MH)), full((1, MH)), full((MH, 1)), full((1, 1)),
                  full((2 * H, 1)), full((1, 1)),
                  full((H, 1)), full((1, 1)),
                  full((H, H)), full((1, H)),
                  col()],
        out_specs=[col(), col(), col(), col(),
                   pl.BlockSpec((BN, H), lambda i: (i, 0)),
                   full((1, 1)), full((1, 1))],
        out_shape=[jax.ShapeDtypeStruct((N, 1), jnp.float32)] * 4
        + [jax.ShapeDtypeStruct((N, H), jnp.float32),
           jax.ShapeDtypeStruct((1, 1), jnp.float32),
           jax.ShapeDtypeStruct((1, 1), jnp.float32)],
    )(psum, p['adj_b2'].reshape(1, -1), p['conf_b2'].reshape(1, -1),
      p['cf_b2'].reshape(1, -1),
      p['mu0_w1'], p['mu0_b1'].reshape(1, -1), p['mu0_w2'],
      p['mu0_b2'].reshape(1, -1),
      p['mu1_w1'], p['mu1_b1'].reshape(1, -1), p['mu1_w2'],
      p['mu1_b2'].reshape(1, -1),
      p['prop_w'], p['prop_b'].reshape(1, -1),
      p['tp_w'], p['tp_b'].reshape(1, -1),
      p['map_w'], p['map_b'].reshape(1, -1),
      t2)


def kernel(x, edge_index, adj_vals, t, params):
    p = params
    row1 = edge_index[0]
    col1 = edge_index[1]

    vals3 = _premask(row1, col1, adj_vals, t)
    ha1, hc1, hf1 = _k1(x, p)
    p1 = _spmm_layer(ha1, hc1, hf1, row1, col1, vals3)
    ha2, hc2, hf2 = _k2(p1, p)
    p2 = _spmm_layer(ha2, hc2, hf2, row1, col1, vals3)
    e2, mu02, mu12, tau2, adj_rep, tl, ml = _k3(p2, t.reshape(N, 1), p)

    e = e2[:, 0]
    mu0 = mu02[:, 0]
    mu1 = mu12[:, 0]
    tau = tau2[:, 0]
    treat_loss = tl[0, 0] / N
    map_loss = ml[0, 0] / (N * H)
    return (e, mu0, mu1, tau, adj_rep, treat_loss, map_loss)


# speedup vs baseline: 1.6758x; 1.0901x over previous
import functools

import jax
import jax.numpy as jnp
from jax import lax
from jax.experimental import pallas as pl
from jax.experimental.pallas import tpu as pltpu
from jax.experimental.pallas import tpu_sc as plsc

N = 10000
E = 320000
D = 128
H = 128
MH = 128

NC = 2
NS = 16
NW = NC * NS
EPW = E // NW
CH = 80
NCHUNK = EPW // CH
SCH = 5
NSUP = NCHUNK // SCH
NP = 10240
RPT = NP // NS
ZR = 32

BN = 1000
GRID = N // BN


def _premask_body(row_h, col_h, vals_h, t_h, out,
                  rowb, colb, valb, vsb, vdb, t_v, sem):
    cid = lax.axis_index("c")
    sid = lax.axis_index("s")
    wid = cid * NS + sid
    pltpu.sync_copy(t_h, t_v)

    def chunk(i, carry):
        base = wid * EPW + i * CH
        c1 = pltpu.async_copy(row_h.at[pl.ds(base, CH)], rowb, sem)
        c2 = pltpu.async_copy(col_h.at[pl.ds(base, CH)], colb, sem)
        c3 = pltpu.async_copy(vals_h.at[pl.ds(base, CH)], valb, sem)
        c1.wait()
        c2.wait()
        c3.wait()
        for s in range(CH // 16):
            sl = pl.ds(s * 16, 16)
            v16 = valb[sl]
            tr = plsc.load_gather(t_v, [rowb[sl]])
            tc = plsc.load_gather(t_v, [colb[sl]])
            vs = jnp.where(tr == tc, v16, 0.0)
            vsb[sl] = vs
            vdb[sl] = v16 - vs
        pltpu.sync_copy(valb, out.at[pl.ds(base, CH)])
        pltpu.sync_copy(vsb, out.at[pl.ds(E + base, CH)])
        pltpu.sync_copy(vdb, out.at[pl.ds(2 * E + base, CH)])
        return carry
    lax.fori_loop(0, NCHUNK, chunk, 0)


def _premask(row2d, col2d, vals2d, t):
    return pl.kernel(
        _premask_body,
        out_type=jax.ShapeDtypeStruct((3 * E,), jnp.float32),
        mesh=plsc.VectorSubcoreMesh(core_axis_name="c", subcore_axis_name="s",
                                    num_cores=NC, num_subcores=NS),
        compiler_params=pltpu.CompilerParams(needs_layout_passes=False),
        scratch_types=[
            pltpu.VMEM((CH,), jnp.int32),
            pltpu.VMEM((CH,), jnp.int32),
            pltpu.VMEM((CH,), jnp.float32),
            pltpu.VMEM((CH,), jnp.float32),
            pltpu.VMEM((CH,), jnp.float32),
            pltpu.VMEM((N,), jnp.float32),
            pltpu.SemaphoreType.DMA,
        ],
    )(row2d, col2d, vals2d, t)


def _spmm_body(ha, hc, hf, row_h, col_h, vals3_h, out,
               rowb, colb, valb, rows0, rows1, zbuf, acc,
               sem_s0, sem_s1, sem_s2, sem_s3, sem_g0, sem_g1, sem_c0, sem_c1):
    cid = lax.axis_index("c")
    sid = lax.axis_index("s")
    wid = cid * NS + sid
    rows = (rows0, rows1)
    sem_s = (sem_s0, sem_s1, sem_s2, sem_s3)
    sem_g = (sem_g0, sem_g1)
    sem_c = (sem_c0, sem_c1)

    def zrow(r, carry):
        for j in range(8):
            zbuf[r, pl.ds(j * 16, 16)] = jnp.zeros((16,), jnp.float32)
        return carry
    lax.fori_loop(0, ZR, zrow, 0)

    def stage(b, c, s):
        base = wid * EPW + c * CH
        pltpu.async_copy(row_h.at[pl.ds(base, CH)], rowb.at[s], sem_s[s])
        pltpu.async_copy(col_h.at[pl.ds(base, CH)], colb.at[s], sem_s[s])
        pltpu.async_copy(vals3_h.at[pl.ds(b * E + base, CH)], valb.at[s],
                         sem_s[s])

    def wait_stage(c, s):
        base = wid * EPW + c * CH
        for _ in range(3):
            pltpu.make_async_copy(row_h.at[pl.ds(base, CH)], rowb.at[s],
                                  sem_s[s]).wait()

    def drain_scatter(r, s):
        pltpu.make_async_copy(rows[r], acc.at[rowb.at[s]], sem_c[r]).wait()

    def do_chunk(h_h, b, c, s, r, drain, more, pre2=True):
        pltpu.make_async_copy(h_h.at[colb.at[s]], rows[r], sem_g[r]).wait()
        if drain:
            drain_scatter(1 - r, (s + 3) % 4)
        if more:
            wait_stage(c + 1, (s + 1) % 4)
            pltpu.async_copy(h_h.at[colb.at[(s + 1) % 4]], rows[1 - r],
                             sem_g[1 - r])
            if pre2:
                stage(b, c + 2, (s + 2) % 4)
        rv = rows[r]

        def edge4(i4, c2_):
            eb = i4 * 4
            sc = jnp.zeros((16,), jnp.int32) + s
            bvs = [plsc.load_gather(
                valb, [sc, jnp.zeros((16,), jnp.int32) + (eb + ee)])
                for ee in range(4)]
            for ee in range(4):
                for jj in range(8):
                    sl2 = pl.ds(jj * 16, 16)
                    rv[eb + ee, sl2] = rv[eb + ee, sl2] * bvs[ee]
            return c2_
        lax.fori_loop(0, CH // 4, edge4, 0)
        pltpu.async_copy(rv, acc.at[rowb.at[s]], sem_c[r], add=True)

    for b, h_h in enumerate((ha, hc, hf)):
        plsc.subcore_barrier()
        for k in range(RPT // ZR):
            pltpu.sync_copy(zbuf, acc.at[pl.ds(sid * RPT + k * ZR, ZR)])
        plsc.subcore_barrier()

        stage(b, 0, 0)
        stage(b, 1, 1)
        wait_stage(0, 0)
        pltpu.async_copy(h_h.at[colb.at[0]], rows0, sem_g0)
        do_chunk(h_h, b, 0, 0, 0, drain=False, more=True)
        do_chunk(h_h, b, 1, 1, 1, drain=True, more=True)

        def quad(q, carry):
            c0 = 2 + 4 * q
            for u in range(4):
                do_chunk(h_h, b, c0 + u, (2 + u) % 4, u % 2,
                         drain=True, more=True)
            return carry
        lax.fori_loop(0, (NCHUNK - 5) // 4, quad, 0)

        do_chunk(h_h, b, NCHUNK - 3, 2, 0, drain=True, more=True, pre2=True)
        do_chunk(h_h, b, NCHUNK - 2, 3, 1, drain=True, more=True, pre2=False)
        do_chunk(h_h, b, NCHUNK - 1, 0, 0, drain=True, more=False)
        drain_scatter(0, 0)

        plsc.subcore_barrier()
        pltpu.sync_copy(acc.at[pl.ds(sid * RPT, RPT)],
                        out.at[b, cid, pl.ds(sid * RPT, RPT)])


def _spmm_layer(ha, hc, hf, row2d, col2d, vals3):
    return pl.kernel(
        _spmm_body,
        out_type=jax.ShapeDtypeStruct((3, NC, NP, H), jnp.float32),
        mesh=plsc.VectorSubcoreMesh(core_axis_name="c", subcore_axis_name="s",
                                    num_cores=NC, num_subcores=NS),
        compiler_params=pltpu.CompilerParams(needs_layout_passes=False),
        scratch_types=[
            pltpu.VMEM((4, CH), jnp.int32),
            pltpu.VMEM((4, CH), jnp.int32),
            pltpu.VMEM((4, CH), jnp.float32),
            pltpu.VMEM((CH, H), jnp.float32),
            pltpu.VMEM((CH, H), jnp.float32),
            pltpu.VMEM((ZR, H), jnp.float32),
            pltpu.VMEM_SHARED((NP, H), jnp.float32),
            pltpu.SemaphoreType.DMA,
            pltpu.SemaphoreType.DMA,
            pltpu.SemaphoreType.DMA,
            pltpu.SemaphoreType.DMA,
            pltpu.SemaphoreType.DMA,
            pltpu.SemaphoreType.DMA,
            pltpu.SemaphoreType.DMA,
            pltpu.SemaphoreType.DMA,
        ],
    )(ha, hc, hf, row2d, col2d, vals3)


def _k1_body(x_ref, dw1, db1, dw2, db2, aw1, cw1, fw1, ha_ref, hc_ref, hf_ref):
    xb = x_ref[...]
    h = jnp.maximum(jnp.dot(xb, dw1[...], preferred_element_type=jnp.float32)
                    + db1[...], 0.0)
    m = jax.nn.sigmoid(jnp.dot(h, dw2[...], preferred_element_type=jnp.float32)
                       + db2[...])
    xa = xb * m
    xc = xb - xa
    ha_ref[...] = jnp.dot(xa, aw1[...], preferred_element_type=jnp.float32)
    hc_ref[...] = jnp.dot(xc, cw1[...], preferred_element_type=jnp.float32)
    hf_ref[...] = jnp.dot(xc, fw1[...], preferred_element_type=jnp.float32)


def _k1(x, p):
    full = lambda s: pl.BlockSpec(s, lambda i: (0, 0))
    row = pl.BlockSpec((BN, D), lambda i: (i, 0))
    return pl.pallas_call(
        _k1_body,
        grid=(GRID,),
        in_specs=[row, full((D, 256)), full((1, 256)), full((256, D)),
                  full((1, D)), full((D, H)), full((D, H)), full((D, H))],
        out_specs=[pl.BlockSpec((BN, H), lambda i: (i, 0))] * 3,
        out_shape=[jax.ShapeDtypeStruct((N, H), jnp.float32)] * 3,
    )(x, p['d_w1'], p['d_b1'].reshape(1, -1), p['d_w2'],
      p['d_b2'].reshape(1, -1), p['adj_w1'], p['conf_w1'], p['cf_w1'])


def _k2_body(p_ref, ba, bc, bf, wa, wc, wf, oa, oc, of):
    for b, (bb, ww, oo) in enumerate(((ba, wa, oa), (bc, wc, oc), (bf, wf, of))):
        s = p_ref[b, 0] + p_ref[b, 1]
        h1 = jnp.maximum(s + bb[...], 0.0)
        oo[...] = jnp.dot(h1, ww[...], preferred_element_type=jnp.float32)


def _k2(psum, p):
    full = lambda s: pl.BlockSpec(s, lambda i: (0, 0))
    return pl.pallas_call(
        _k2_body,
        grid=(GRID,),
        in_specs=[pl.BlockSpec((3, NC, BN, H), lambda i: (0, 0, i, 0)),
                  full((1, H)), full((1, H)), full((1, H)),
                  full((H, H)), full((H, H)), full((H, H))],
        out_specs=[pl.BlockSpec((BN, H), lambda i: (i, 0))] * 3,
        out_shape=[jax.ShapeDtypeStruct((N, H), jnp.float32)] * 3,
    )(psum, p['adj_b1'].reshape(1, -1), p['conf_b1'].reshape(1, -1),
      p['cf_b1'].reshape(1, -1), p['adj_w2'], p['conf_w2'], p['cf_w2'])


def _k3_body(p_ref, ba, bc, bf, m0w1, m0b1, m0w2, m0b2, m1w1, m1b1, m1w2,
             m1b2, pw, pb, tpw, tpb, mw, mb, t_ref,
             e_ref, mu0_ref, mu1_ref, tau_ref, adj_ref, tl_ref, ml_ref):
    i = pl.program_id(0)
    adj = jnp.maximum(p_ref[0, 0] + p_ref[0, 1] + ba[...], 0.0)
    conf = jnp.maximum(p_ref[1, 0] + p_ref[1, 1] + bc[...], 0.0)
    cf = jnp.maximum(p_ref[2, 0] + p_ref[2, 1] + bf[...], 0.0)
    adj_ref[...] = adj

    def cat_mm(w_ref):
        return (jnp.dot(adj, w_ref[0:H, :], preferred_element_type=jnp.float32)
                + jnp.dot(conf, w_ref[H:2 * H, :],
                          preferred_element_type=jnp.float32))

    h0 = jnp.maximum(cat_mm(m0w1) + m0b1[...], 0.0)
    mu0 = jnp.dot(h0, m0w2[...], preferred_element_type=jnp.float32) + m0b2[...]
    h1 = jnp.maximum(cat_mm(m1w1) + m1b1[...], 0.0)
    mu1 = jnp.dot(h1, m1w2[...], preferred_element_type=jnp.float32) + m1b2[...]
    ev = jax.nn.sigmoid(cat_mm(pw) + pb[...])
    e_ref[...] = ev
    mu0_ref[...] = mu0
    mu1_ref[...] = mu1
    tau_ref[...] = mu1 - mu0

    tp = jax.nn.sigmoid(jnp.dot(conf, tpw[...],
                                preferred_element_type=jnp.float32) + tpb[...])
    eps = 1e-7
    tpc = jnp.clip(tp, eps, 1.0 - eps)
    tb = t_ref[...]
    bce = -(tb * jnp.log(tpc) + (1.0 - tb) * jnp.log(1.0 - tpc))
    mapped = (jnp.dot(conf, mw[...], preferred_element_type=jnp.float32)
              + mb[...])
    se = (mapped - cf) ** 2

    @pl.when(i == 0)
    def _():
        tl_ref[...] = jnp.zeros((1, 1), jnp.float32)
        ml_ref[...] = jnp.zeros((1, 1), jnp.float32)
    tl_ref[...] += jnp.sum(bce).reshape(1, 1)
    ml_ref[...] += jnp.sum(se).reshape(1, 1)


def _k3(psum, t2, p):
    full = lambda s: pl.BlockSpec(s, lambda i: (0, 0))
    col = lambda: pl.BlockSpec((BN, 1), lambda i: (i, 0))
    return pl.pallas_call(
        _k3_body,
        grid=(GRID,),
        in_specs=[pl.BlockSpec((3, NC, BN, H), lambda i: (0, 0, i, 0)),
                  full((1, H)), full((1, H)), full((1, H)),
                  full((2 * H, MH)), full((1, MH)), full((MH, 1)), full((1, 1)),
                  full((2 * H, MH)), full((1, MH)), full((MH, 1)), full((1, 1)),
                  full((2 * H, 1)), full((1, 1)),
                  full((H, 1)), full((1, 1)),
                  full((H, H)), full((1, H)),
                  col()],
        out_specs=[col(), col(), col(), col(),
                   pl.BlockSpec((BN, H), lambda i: (i, 0)),
                   full((1, 1)), full((1, 1))],
        out_shape=[jax.ShapeDtypeStruct((N, 1), jnp.float32)] * 4
        + [jax.ShapeDtypeStruct((N, H), jnp.float32),
           jax.ShapeDtypeStruct((1, 1), jnp.float32),
           jax.ShapeDtypeStruct((1, 1), jnp.float32)],
    )(psum, p['adj_b2'].reshape(1, -1), p['conf_b2'].reshape(1, -1),
      p['cf_b2'].reshape(1, -1),
      p['mu0_w1'], p['mu0_b1'].reshape(1, -1), p['mu0_w2'],
      p['mu0_b2'].reshape(1, -1),
      p['mu1_w1'], p['mu1_b1'].reshape(1, -1), p['mu1_w2'],
      p['mu1_b2'].reshape(1, -1),
      p['prop_w'], p['prop_b'].reshape(1, -1),
      p['tp_w'], p['tp_b'].reshape(1, -1),
      p['map_w'], p['map_b'].reshape(1, -1),
      t2)


def kernel(x, edge_index, adj_vals, t, params):
    p = params
    row1 = edge_index[0]
    col1 = edge_index[1]

    vals3 = _premask(row1, col1, adj_vals, t)
    ha1, hc1, hf1 = _k1(x, p)
    p1 = _spmm_layer(ha1, hc1, hf1, row1, col1, vals3)
    ha2, hc2, hf2 = _k2(p1, p)
    p2 = _spmm_layer(ha2, hc2, hf2, row1, col1, vals3)
    e2, mu02, mu12, tau2, adj_rep, tl, ml = _k3(p2, t.reshape(N, 1), p)

    e = e2[:, 0]
    mu0 = mu02[:, 0]
    mu1 = mu12[:, 0]
    tau = tau2[:, 0]
    treat_loss = tl[0, 0] / N
    map_loss = ml[0, 0] / (N * H)
    return (e, mu0, mu1, tau, adj_rep, treat_loss, map_loss)
